# Initial kernel scaffold; baseline (speedup 1.0000x reference)
import functools

import jax
import jax.numpy as jnp
from jax import lax
from jax.experimental import pallas as pl
from jax.experimental.pallas import tpu as pltpu
from jax.experimental.pallas import tpu_sc as plsc

N = 10000
E = 160000
D = 256
K = 8
H = 256
R = 50

HH = H // 2
KW = 16
NC = 2
NS = 16
EPT = E // NS
C1 = 80
C2 = 40
NCH1 = EPT // C1
NCH2 = EPT // C2
NPT = N // NS
NB = 1000
GRID_N = N // NB

_SLOPE = (1.0 / 8.0 + 1.0 / 3.0) / 2.0


def _precompute_body(nf_ref, rf_ref, wm_ref, att_ref, wml_ref, wmh_ref,
                     rw_ref, ssrc_ref, srel_ref, yrl_ref, yrh_ref, rout_ref):
  wm = wm_ref[...]
  att = att_ref[...]
  watt = jnp.sum(wm * att[None, :, :], axis=2)
  watt16 = jnp.concatenate(
      [watt, jnp.zeros((D, KW - K), jnp.float32)], axis=1)
  nf = nf_ref[...]
  rf = rf_ref[...]
  ssrc_ref[...] = jnp.dot(nf, watt16, preferred_element_type=jnp.float32)
  srel_ref[...] = jnp.dot(rf, watt16, preferred_element_type=jnp.float32)
  yrl_ref[...] = jnp.dot(rf, wml_ref[...], preferred_element_type=jnp.float32)
  yrh_ref[...] = jnp.dot(rf, wmh_ref[...], preferred_element_type=jnp.float32)
  rout_ref[...] = jnp.dot(rf, rw_ref[...], preferred_element_type=jnp.float32)


def _precompute(nf, rf, wm, att, wml, wmh, rw):
  return pl.pallas_call(
      _precompute_body,
      out_shape=(
          jax.ShapeDtypeStruct((N, KW), jnp.float32),
          jax.ShapeDtypeStruct((R, KW), jnp.float32),
          jax.ShapeDtypeStruct((R, K * HH), jnp.float32),
          jax.ShapeDtypeStruct((R, K * HH), jnp.float32),
          jax.ShapeDtypeStruct((R, H), jnp.float32),
      ),
  )(nf, rf, wm, att, wml, wmh, rw)


def _ytables_body(nf_ref, wml_ref, wmh_ref, yl_ref, yh_ref):
  nf = nf_ref[...]
  yl_ref[...] = jnp.dot(nf, wml_ref[...], preferred_element_type=jnp.float32)
  yh_ref[...] = jnp.dot(nf, wmh_ref[...], preferred_element_type=jnp.float32)


def _ytables(nf, wml, wmh):
  return pl.pallas_call(
      _ytables_body,
      grid=(GRID_N,),
      in_specs=[
          pl.BlockSpec((NB, D), lambda i: (i, 0)),
          pl.BlockSpec((D, K * HH), lambda i: (0, 0)),
          pl.BlockSpec((D, K * HH), lambda i: (0, 0)),
      ],
      out_specs=(
          pl.BlockSpec((NB, K * HH), lambda i: (i, 0)),
          pl.BlockSpec((NB, K * HH), lambda i: (i, 0)),
      ),
      out_shape=(
          jax.ShapeDtypeStruct((N, K * HH), jnp.float32),
          jax.ShapeDtypeStruct((N, K * HH), jnp.float32),
      ),
  )(nf, wml, wmh)


def _sc_body(ssrc_hbm, srel_hbm, yl_hbm, yh_hbm, yrl_hbm, yrh_hbm,
             src_hbm, dst_hbm, et_hbm, outl_hbm, outh_hbm,
             yg, yrg, zb, sg1, sr1, pb, sg2, sr2, dng,
             si1, di1, ei1, si2, di2, ei2, zrow, denom_sp, agg_sp):
  cid = lax.axis_index("c")
  sid = lax.axis_index("s")

  @pl.loop(0, 125)
  def _(r):
    for hh in range(HH // 16):
      zrow[r, pl.ds(hh * 16, 16)] = jnp.zeros((16,), jnp.float32)

  @pl.loop(0, NPT // 125)
  def _(b):
    pltpu.sync_copy(zrow, agg_sp.at[pl.ds(sid * NPT + b * 125, 125), :])

  @pl.loop(0, NPT // 125)
  def _(b):
    pltpu.sync_copy(zrow.at[:, pl.ds(0, KW)],
                    denom_sp.at[pl.ds(sid * NPT + b * 125, 125), :])

  plsc.subcore_barrier()

  ebase = sid * EPT

  @pl.loop(0, NCH1)
  def _(j):
    base = ebase + j * C1
    pltpu.sync_copy(src_hbm.at[pl.ds(base, C1)], si1)
    pltpu.sync_copy(dst_hbm.at[pl.ds(base, C1)], di1)
    pltpu.sync_copy(et_hbm.at[pl.ds(base, C1)], ei1)
    pltpu.sync_copy(ssrc_hbm.at[si1], sg1)
    pltpu.sync_copy(srel_hbm.at[ei1], sr1)

    @pl.loop(0, C1)
    def _(e):
      s = sg1[e, :] + sr1[e, :]
      s = jnp.where(s >= 0, s, 0.01 * s)
      pb[e, :] = jnp.exp(s)

    pltpu.sync_copy(pb, denom_sp.at[di1], add=True)

  plsc.subcore_barrier()

  @pl.loop(0, NCH2)
  def _(j):
    base = ebase + j * C2
    pltpu.sync_copy(src_hbm.at[pl.ds(base, C2)], si2)
    pltpu.sync_copy(dst_hbm.at[pl.ds(base, C2)], di2)
    pltpu.sync_copy(et_hbm.at[pl.ds(base, C2)], ei2)
    pltpu.sync_copy(ssrc_hbm.at[si2], sg2)
    pltpu.sync_copy(srel_hbm.at[ei2], sr2)
    pltpu.sync_copy(denom_sp.at[di2], dng)

    @pl.when(cid == 0)
    def _():
      pltpu.sync_copy(yl_hbm.at[si2], yg)
      pltpu.sync_copy(yrl_hbm.at[ei2], yrg)

    @pl.when(cid == 1)
    def _():
      pltpu.sync_copy(yh_hbm.at[si2], yg)
      pltpu.sync_copy(yrh_hbm.at[ei2], yrg)

    @pl.loop(0, C2)
    def _(e):
      s = sg2[e, :] + sr2[e, :]
      s = jnp.where(s >= 0, s, 0.01 * s)
      wv = jnp.exp(s) / dng[e, :]
      for hh in range(HH // 16):
        acc = jnp.zeros((16,), jnp.float32)
        for k in range(K):
          yv = yg[e, pl.ds(k * HH + hh * 16, 16)]
          yrv = yrg[e, pl.ds(k * HH + hh * 16, 16)]
          acc = acc + wv[k] * (yv + yrv)
        zb[e, pl.ds(hh * 16, 16)] = acc

    pltpu.sync_copy(zb, agg_sp.at[di2], add=True)

  plsc.subcore_barrier()

  @pl.when(cid == 0)
  def _():
    pltpu.sync_copy(agg_sp.at[pl.ds(sid * NPT, NPT), :],
                    outl_hbm.at[pl.ds(sid * NPT, NPT), :])

  @pl.when(cid == 1)
  def _():
    pltpu.sync_copy(agg_sp.at[pl.ds(sid * NPT, NPT), :],
                    outh_hbm.at[pl.ds(sid * NPT, NPT), :])


def _sc_aggregate(ssrc, srel, yl, yh, yrl, yrh, src, dst, et):
  mesh = plsc.VectorSubcoreMesh(
      core_axis_name="c", subcore_axis_name="s", num_cores=NC,
      num_subcores=NS)
  kern = pl.kernel(
      _sc_body,
      out_type=(
          jax.ShapeDtypeStruct((N, HH), jnp.float32),
          jax.ShapeDtypeStruct((N, HH), jnp.float32),
      ),
      mesh=mesh,
      scratch_types=[
          pltpu.VMEM((C2, K * HH), jnp.float32),
          pltpu.VMEM((C2, K * HH), jnp.float32),
          pltpu.VMEM((C2, HH), jnp.float32),
          pltpu.VMEM((C1, KW), jnp.float32),
          pltpu.VMEM((C1, KW), jnp.float32),
          pltpu.VMEM((C1, KW), jnp.float32),
          pltpu.VMEM((C2, KW), jnp.float32),
          pltpu.VMEM((C2, KW), jnp.float32),
          pltpu.VMEM((C2, KW), jnp.float32),
          pltpu.VMEM((C1,), jnp.int32),
          pltpu.VMEM((C1,), jnp.int32),
          pltpu.VMEM((C1,), jnp.int32),
          pltpu.VMEM((C2,), jnp.int32),
          pltpu.VMEM((C2,), jnp.int32),
          pltpu.VMEM((C2,), jnp.int32),
          pltpu.VMEM((125, HH), jnp.float32),
          pltpu.VMEM_SHARED((N, KW), jnp.float32),
          pltpu.VMEM_SHARED((N, HH), jnp.float32),
      ],
  )
  return kern(ssrc, srel, yl, yh, yrl, yrh, src, dst, et)


def _combine_body(al_ref, ah_ref, nf_ref, lw_ref, out_ref):
  sm = jnp.dot(nf_ref[...], lw_ref[...], preferred_element_type=jnp.float32)
  inm = jnp.concatenate([al_ref[...], ah_ref[...]], axis=1) * (1.0 / K)
  y = sm + inm
  out_ref[...] = jnp.where(y >= 0, y, _SLOPE * y)


def _combine(al, ah, nf, lw):
  return pl.pallas_call(
      _combine_body,
      grid=(GRID_N,),
      in_specs=[
          pl.BlockSpec((NB, HH), lambda i: (i, 0)),
          pl.BlockSpec((NB, HH), lambda i: (i, 0)),
          pl.BlockSpec((NB, D), lambda i: (i, 0)),
          pl.BlockSpec((D, H), lambda i: (0, 0)),
      ],
      out_specs=pl.BlockSpec((NB, H), lambda i: (i, 0)),
      out_shape=jax.ShapeDtypeStruct((N, H), jnp.float32),
  )(al, ah, nf, lw)


def kernel(node_feats, rel_feats, msg_weight, att_weight, loop_weight,
           rel_weight, edge_index, edge_types):
  src = edge_index[0]
  dst = edge_index[1]
  wml = msg_weight[:, :, :HH].reshape(D, K * HH)
  wmh = msg_weight[:, :, HH:].reshape(D, K * HH)

  ssrc, srel, yrl, yrh, rel_out = _precompute(
      node_feats, rel_feats, msg_weight, att_weight, wml, wmh, rel_weight)
  yl, yh = _ytables(node_feats, wml, wmh)
  agg_lo, agg_hi = _sc_aggregate(ssrc, srel, yl, yh, yrl, yrh, src, dst,
                                 edge_types)
  agg_msg = _combine(agg_lo, agg_hi, node_feats, loop_weight)
  return (agg_msg, rel_out)


# speedup vs baseline: 2.1511x; 2.1511x over previous
import jax
import jax.numpy as jnp
from jax import lax
from jax.experimental import pallas as pl
from jax.experimental.pallas import tpu as pltpu
from jax.experimental.pallas import tpu_sc as plsc

N = 10000
E = 160000
D = 256
K = 8
H = 256
R = 50

HH = H // 2
KW = 16
SW = 128
NC = 2
NS = 16
NT = NC * K
EPT = E // NS
C = 40
NCH = EPT // C
ND8 = N // 8
STRIPE = 640
LAST_STRIPE = N - STRIPE * (NS - 1)
DSTRIPE = 80
NB = 1000
GRID_N = N // NB

_SLOPE = (1.0 / 8.0 + 1.0 / 3.0) / 2.0


def _precompute_body(nf_ref, rf_ref, wm_ref, att_ref, rw_ref,
                     ssrc_ref, srel_ref, rout_ref, *yr_refs):
  wm = wm_ref[...]
  att = att_ref[...]
  watt = jnp.sum(wm * att[None, :, :], axis=2)
  watt_pad = jnp.concatenate(
      [watt, jnp.zeros((D, SW - K), jnp.float32)], axis=1)
  nf = nf_ref[...]
  rf = rf_ref[...]
  ssrc_ref[...] = jnp.dot(nf, watt_pad, preferred_element_type=jnp.float32)
  srel_ref[...] = jnp.dot(rf, watt_pad, preferred_element_type=jnp.float32)
  rout_ref[...] = jnp.dot(rf, rw_ref[...], preferred_element_type=jnp.float32)
  for t in range(NT):
    half, k = divmod(t, K)
    wq = wm[:, k, half * HH:(half + 1) * HH]
    yr_refs[t][...] = jnp.dot(rf, wq, preferred_element_type=jnp.float32)


def _precompute(nf, rf, wm, att, rw):
  return pl.pallas_call(
      _precompute_body,
      out_shape=(
          jax.ShapeDtypeStruct((N, SW), jnp.float32),
          jax.ShapeDtypeStruct((R, SW), jnp.float32),
          jax.ShapeDtypeStruct((R, H), jnp.float32),
      ) + tuple(
          jax.ShapeDtypeStruct((R, HH), jnp.float32) for _ in range(NT)
      ),
  )(nf, rf, wm, att, rw)


def _ytables_body(nf_ref, *refs):
  wq_refs = refs[:NT]
  y_refs = refs[NT:]
  nf = nf_ref[...]
  for t in range(NT):
    y_refs[t][...] = jnp.dot(nf, wq_refs[t][...],
                             preferred_element_type=jnp.float32)


def _ytables(nf, wqs):
  return pl.pallas_call(
      _ytables_body,
      grid=(GRID_N,),
      in_specs=[pl.BlockSpec((NB, D), lambda i: (i, 0))] + [
          pl.BlockSpec((D, HH), lambda i: (0, 0)) for _ in range(NT)
      ],
      out_specs=tuple(
          pl.BlockSpec((NB, HH), lambda i: (i, 0)) for _ in range(NT)
      ),
      out_shape=tuple(
          jax.ShapeDtypeStruct((N, HH), jnp.float32) for _ in range(NT)
      ),
  )(nf, *wqs)


def _sc_body(*refs):
  (ssrc_hbm, srel_hbm) = refs[:2]
  y_tabs = refs[2:2 + NT]
  yr_tabs = refs[2 + NT:2 + 2 * NT]
  (src_hbm, dst_hbm, et_hbm) = refs[2 + 2 * NT:5 + 2 * NT]
  (o0, o1, w0_hbm, w1_hbm) = refs[5 + 2 * NT:9 + 2 * NT]
  (yg, yrg, zb, wb, si, di, ei, di8, denom_sp, agg_sp, dsem) = (
      refs[9 + 2 * NT:])
  sg, sr = yg, yrg
  cid = lax.axis_index("c")
  sid = lax.axis_index("s")
  ebase = sid * EPT
  nzb = jnp.where(sid == NS - 1, LAST_STRIPE // C, STRIPE // C)

  def cp(sref, dref, add=False):
    pltpu.async_copy(sref, dref, dsem, add=add).wait()

  @pl.loop(0, C)
  def _(r):
    for hh in range(HH // 16):
      zb[r, pl.ds(hh * 16, 16)] = jnp.zeros((16,), jnp.float32)

  @pl.loop(0, nzb)
  def _(b):
    cp(zb, agg_sp.at[pl.ds(sid * STRIPE + b * C, C), :])

  @pl.when(sid < NS - 1)
  def _():
    cp(zb, denom_sp.at[pl.ds(sid * DSTRIPE, C), :])
    cp(zb, denom_sp.at[pl.ds(sid * DSTRIPE + C, C), :])

  @pl.when(sid == NS - 1)
  def _():
    cp(zb.at[pl.ds(0, 25), :],
       denom_sp.at[pl.ds((NS - 1) * DSTRIPE, 25), :])
    cp(zb.at[pl.ds(0, 25), :],
       denom_sp.at[pl.ds((NS - 1) * DSTRIPE + 25, 25), :])

  plsc.subcore_barrier()

  def load_chunk_indices(base):
    cp(src_hbm.at[pl.ds(base, C)], si)
    cp(dst_hbm.at[pl.ds(base, C)], di)
    cp(et_hbm.at[pl.ds(base, C)], ei)

  def dvecs():
    return (di[pl.ds(0, 16)], di[pl.ds(16, 16)], di[pl.ds(24, 16)])

  def dscalar(dv, e):
    if e < 16:
      return dv[0][e]
    if e < 32:
      return dv[1][e - 16]
    return dv[2][e - 24]

  def compute_di8():
    v0 = di[pl.ds(0, 16)] >> 3
    v1 = di[pl.ds(16, 16)] >> 3
    v2 = di[pl.ds(24, 16)] >> 3
    di8[pl.ds(0, 16)] = v0
    di8[pl.ds(16, 16)] = v1
    di8[pl.ds(24, 16)] = v2

  def pvec(e):
    s = sg[e, pl.ds(0, KW)] + sr[e, pl.ds(0, KW)]
    s = jnp.where(s >= 0, s, 0.01 * s)
    return jnp.exp(s)

  @pl.loop(0, NCH)
  def _(j):
    base = ebase + j * C
    load_chunk_indices(base)
    cp(ssrc_hbm.at[si], sg)
    cp(srel_hbm.at[ei], sr)
    compute_di8()
    dv = dvecs()
    for e in range(C):
      col = (dscalar(dv, e) & 7) * KW
      for hh in range(HH // 16):
        zb[e, pl.ds(hh * 16, 16)] = jnp.zeros((16,), jnp.float32)
      zb[e, pl.ds(col, KW)] = pvec(e)
    cp(zb, denom_sp.at[di8], add=True)

  plsc.subcore_barrier()

  @pl.loop(0, NCH)
  def _(j):
    base = ebase + j * C
    load_chunk_indices(base)
    cp(ssrc_hbm.at[si], sg)
    cp(srel_hbm.at[ei], sr)
    compute_di8()
    cp(denom_sp.at[di8], zb)
    dv = dvecs()
    for e in range(C):
      col = (dscalar(dv, e) & 7) * KW
      wb[e, :] = pvec(e) / zb[e, pl.ds(col, KW)]

    @pl.when(cid == 0)
    def _():
      cp(wb, w0_hbm.at[pl.ds(base, C), :])

    @pl.when(cid == 1)
    def _():
      cp(wb, w1_hbm.at[pl.ds(base, C), :])

  plsc.subcore_barrier()

  ws = [w0_hbm, w1_hbm]
  outs = [o0, o1]
  for core in range(NC):
    @pl.when(cid == core)
    def _(core=core):
      for k in range(K):
        t = core * K + k

        @pl.loop(0, NCH)
        def _(j, t=t, k=k):
          base = ebase + j * C
          cp(src_hbm.at[pl.ds(base, C)], si)
          cp(dst_hbm.at[pl.ds(base, C)], di)
          cp(et_hbm.at[pl.ds(base, C)], ei)
          cp(ws[core].at[pl.ds(base, C), :], wb)
          cp(y_tabs[t].at[si], yg)
          cp(yr_tabs[t].at[ei], yrg)

          @pl.loop(0, C)
          def _(e):
            wk = wb[e, :][k]
            for hh in range(HH // 16):
              yv = yg[e, pl.ds(hh * 16, 16)]
              yrv = yrg[e, pl.ds(hh * 16, 16)]
              zb[e, pl.ds(hh * 16, 16)] = wk * (yv + yrv)

          cp(zb, agg_sp.at[di], add=True)

      plsc.subcore_barrier()

      @pl.when(sid < NS - 1)
      def _():
        cp(agg_sp.at[pl.ds(sid * STRIPE, STRIPE), :],
           outs[core].at[pl.ds(sid * STRIPE, STRIPE), :])

      @pl.when(sid == NS - 1)
      def _():
        cp(agg_sp.at[pl.ds((NS - 1) * STRIPE, LAST_STRIPE), :],
           outs[core].at[pl.ds((NS - 1) * STRIPE, LAST_STRIPE), :])


def _sc_aggregate(ssrc, srel, ys, yrs, src, dst, et):
  mesh = plsc.VectorSubcoreMesh(
      core_axis_name="c", subcore_axis_name="s", num_cores=NC,
      num_subcores=NS)
  kern = pl.kernel(
      _sc_body,
      out_type=(
          jax.ShapeDtypeStruct((N, HH), jnp.float32),
          jax.ShapeDtypeStruct((N, HH), jnp.float32),
          jax.ShapeDtypeStruct((E, KW), jnp.float32),
          jax.ShapeDtypeStruct((E, KW), jnp.float32),
      ),
      mesh=mesh,
      scratch_types=[
          pltpu.VMEM((C, HH), jnp.float32),
          pltpu.VMEM((C, HH), jnp.float32),
          pltpu.VMEM((C, HH), jnp.float32),
          pltpu.VMEM((C, KW), jnp.float32),
          pltpu.VMEM((C,), jnp.int32),
          pltpu.VMEM((C,), jnp.int32),
          pltpu.VMEM((C,), jnp.int32),
          pltpu.VMEM((C,), jnp.int32),
          pltpu.VMEM_SHARED((ND8, 128), jnp.float32),
          pltpu.VMEM_SHARED((N, HH), jnp.float32),
          pltpu.SemaphoreType.DMA,
      ],
  )
  return kern(ssrc, srel, *ys, *yrs, src, dst, et)


def _combine_body(a0, a1, nf_ref, lw_ref, out_ref):
  sm = jnp.dot(nf_ref[...], lw_ref[...], preferred_element_type=jnp.float32)
  inm = jnp.concatenate([a0[...], a1[...]], axis=1) * (1.0 / K)
  y = sm + inm
  out_ref[...] = jnp.where(y >= 0, y, _SLOPE * y)


def _combine(aggs, nf, lw):
  return pl.pallas_call(
      _combine_body,
      grid=(GRID_N,),
      in_specs=[
          pl.BlockSpec((NB, HH), lambda i: (i, 0)),
          pl.BlockSpec((NB, HH), lambda i: (i, 0)),
          pl.BlockSpec((NB, D), lambda i: (i, 0)),
          pl.BlockSpec((D, H), lambda i: (0, 0)),
      ],
      out_specs=pl.BlockSpec((NB, H), lambda i: (i, 0)),
      out_shape=jax.ShapeDtypeStruct((N, H), jnp.float32),
  )(*aggs, nf, lw)


def kernel(node_feats, rel_feats, msg_weight, att_weight, loop_weight,
           rel_weight, edge_index, edge_types):
  src = edge_index[0]
  dst = edge_index[1]
  wqs = []
  for t in range(NT):
    half, k = divmod(t, K)
    wqs.append(msg_weight[:, k, half * HH:(half + 1) * HH])

  ssrc, srel, rel_out, *yrs = _precompute(
      node_feats, rel_feats, msg_weight, att_weight, rel_weight)
  ys = _ytables(node_feats, wqs)
  a0, a1, _w0, _w1 = _sc_aggregate(
      ssrc, srel, ys, yrs, src, dst, edge_types)
  agg_msg = _combine([a0, a1], node_feats, loop_weight)
  return (agg_msg, rel_out)
